# baseline (speedup 1.0000x reference)
import functools

import jax
import jax.numpy as jnp
from jax import lax
from jax.experimental import pallas as pl
from jax.experimental.pallas import tpu as pltpu
from jax.experimental.pallas import tpu_sc as plsc


@functools.lru_cache(maxsize=None)
def _make_flip(L: int, H: int):
    info = plsc.get_sparse_core_info()
    NC, NS, LANES = info.num_cores, info.num_subcores, info.num_lanes
    NW = NC * NS
    rows_per_w = L // NW
    C = 32
    NBUF = 3
    n_chunks = rows_per_w // C

    mesh = plsc.VectorSubcoreMesh(core_axis_name="c", subcore_axis_name="s")

    @functools.partial(
        pl.kernel,
        mesh=mesh,
        out_type=jax.ShapeDtypeStruct((L, H), jnp.float32),
        scratch_types=[
            pltpu.VMEM((n_chunks, C), jnp.int32),
            pltpu.VMEM((NBUF, C, H), jnp.float32),
            pltpu.SemaphoreType.DMA,
            pltpu.SemaphoreType.DMA,
            pltpu.SemaphoreType.DMA,
            pltpu.SemaphoreType.DMA,
            pltpu.SemaphoreType.DMA,
            pltpu.SemaphoreType.DMA,
        ],
    )
    def flip_k(table_hbm, out_hbm, idx_v, buf_v, g0, g1, g2, w0, w1, w2):
        gsem = (g0, g1, g2)
        wsem = (w0, w1, w2)
        wid = lax.axis_index("s") * NC + lax.axis_index("c")

        def chunk_base(c):
            return (c * NW + wid) * C

        for c in range(n_chunks):
            top = (L - 1) - chunk_base(c)
            for i in range(C // LANES):
                idx_v[c, pl.ds(i * LANES, LANES)] = (
                    (top - i * LANES) - lax.iota(jnp.int32, LANES)
                )

        def gather(c):
            b = c % NBUF
            return pltpu.async_copy(
                table_hbm.at[idx_v.at[c]], buf_v.at[b], gsem[b]
            )

        gops, wops = {}, {}
        for c in range(min(2, n_chunks)):
            gops[c] = gather(c)
        for c in range(n_chunks):
            b = c % NBUF
            gops[c].wait()
            wops[c] = pltpu.async_copy(
                buf_v.at[b], out_hbm.at[pl.ds(chunk_base(c), C)], wsem[b]
            )
            if c + 2 < n_chunks:
                if c >= 1:
                    wops[c - 1].wait()
                gops[c + 2] = gather(c + 2)
        wops[n_chunks - 3].wait()
        wops[n_chunks - 2].wait()
        wops[n_chunks - 1].wait()

    return flip_k


def kernel(hidden_states, pos_table):
    L = hidden_states.shape[1]
    H = pos_table.shape[1]
    out = _make_flip(L, H)(pos_table)
    return out.reshape(1, L, H)
